# baseline (speedup 1.0000x reference)
import functools

import jax
import jax.numpy as jnp
from jax import lax
from jax.experimental import pallas as pl
from jax.experimental.pallas import tpu as pltpu
from jax.experimental.pallas import tpu_sc as plsc

BATCH = 16384
NUM_FIELDS = 26
NUM_EMB = 2600000
TBL_SPLIT = 2599936
NUM_WORKERS = 32
ROWS_PER_W = BATCH // NUM_WORKERS
IDX_PER_W = ROWS_PER_W * NUM_FIELDS
F_HALF = NUM_FIELDS // 2
IDX_HALF = F_HALF * ROWS_PER_W


@functools.partial(
    pl.kernel,
    out_type=jax.ShapeDtypeStruct((BATCH,), jnp.float32),
    mesh=plsc.VectorSubcoreMesh(core_axis_name="c", subcore_axis_name="s"),
    scratch_types=[
        pltpu.VMEM((IDX_PER_W,), jnp.int32),
        pltpu.VMEM((IDX_PER_W,), jnp.float32),
        pltpu.VMEM((ROWS_PER_W,), jnp.float32),
        pltpu.VMEM((16,), jnp.float32),
        pltpu.SemaphoreType.DMA,
        pltpu.SemaphoreType.DMA,
        pltpu.SemaphoreType.DMA,
    ],
)
def _emb_sum(x_hbm, table_hbm, bias_hbm, out_hbm, idx_v, vals_v, out_v, bias_v,
             sem1, sem2, sem3):
    wid = lax.axis_index("s") * 2 + lax.axis_index("c")

    col = wid * ROWS_PER_W
    for f in range(NUM_FIELDS):
        pltpu.async_copy(
            x_hbm.at[f, pl.ds(col, ROWS_PER_W)],
            idx_v.at[pl.ds(f * ROWS_PER_W, ROWS_PER_W)],
            sem3,
        )
    pltpu.sync_copy(bias_hbm, bias_v)
    bias_vec = bias_v[...]
    for f in range(NUM_FIELDS):
        pltpu.make_async_copy(
            x_hbm.at[f, pl.ds(col, ROWS_PER_W)],
            idx_v.at[pl.ds(f * ROWS_PER_W, ROWS_PER_W)],
            sem3,
        ).wait()
        pltpu.async_copy(
            table_hbm.at[idx_v.at[pl.ds(f * ROWS_PER_W, ROWS_PER_W)]],
            vals_v.at[pl.ds(f * ROWS_PER_W, ROWS_PER_W)],
            sem1 if f % 2 == 0 else sem2,
        )

    def init_out(i, _):
        out_v[pl.ds(i * 16, 16)] = bias_vec
        return 0

    lax.fori_loop(0, ROWS_PER_W // 16, init_out, 0)

    for f in range(NUM_FIELDS):
        pltpu.make_async_copy(
            table_hbm.at[idx_v.at[pl.ds(f * ROWS_PER_W, ROWS_PER_W)]],
            vals_v.at[pl.ds(f * ROWS_PER_W, ROWS_PER_W)],
            sem1 if f % 2 == 0 else sem2,
        ).wait()

        def faccum(i, _, f=f):
            out_v[pl.ds(i * 16, 16)] = (
                out_v[pl.ds(i * 16, 16)]
                + vals_v[pl.ds(f * ROWS_PER_W + i * 16, 16)]
            )
            return 0

        lax.fori_loop(0, ROWS_PER_W // 16, faccum, 0)

    pltpu.sync_copy(out_v, out_hbm.at[pl.ds(wid * ROWS_PER_W, ROWS_PER_W)])


def kernel(x, table, bias):
    table_lin = jnp.concatenate(
        [
            table[:TBL_SPLIT].reshape(-1),
            table[TBL_SPLIT:].reshape(-1),
        ]
    )
    xw = x.T
    out = _emb_sum(xw, table_lin, jnp.broadcast_to(bias, (16,)))
    return out.reshape(BATCH, 1)


# speedup vs baseline: 1.0840x; 1.0840x over previous
import functools

import jax
import jax.numpy as jnp
from jax import lax
from jax.experimental import pallas as pl
from jax.experimental.pallas import tpu as pltpu
from jax.experimental.pallas import tpu_sc as plsc

BATCH = 16384
NUM_FIELDS = 26
NUM_EMB = 2600000
TBL_SPLIT = 2599936
NUM_WORKERS = 32
ROWS_PER_W = BATCH // NUM_WORKERS
IDX_PER_W = ROWS_PER_W * NUM_FIELDS
F_HALF = NUM_FIELDS // 2
IDX_HALF = F_HALF * ROWS_PER_W


@functools.partial(
    pl.kernel,
    out_type=jax.ShapeDtypeStruct((BATCH,), jnp.float32),
    mesh=plsc.VectorSubcoreMesh(core_axis_name="c", subcore_axis_name="s"),
    scratch_types=[
        pltpu.VMEM((IDX_PER_W,), jnp.int32),
        pltpu.VMEM((IDX_PER_W,), jnp.float32),
        pltpu.VMEM((ROWS_PER_W,), jnp.float32),
        pltpu.VMEM((16,), jnp.float32),
        pltpu.SemaphoreType.DMA,
        pltpu.SemaphoreType.DMA,
        pltpu.SemaphoreType.DMA,
    ],
)
def _emb_sum(x_hbm, table_hbm, bias_hbm, out_hbm, idx_v, vals_v, out_v, bias_v,
             sem1, sem2, sem3):
    wid = lax.axis_index("s") * 2 + lax.axis_index("c")

    col = wid * ROWS_PER_W
    for f in range(F_HALF):
        pltpu.async_copy(
            x_hbm.at[f, pl.ds(col, ROWS_PER_W)],
            idx_v.at[pl.ds(f * ROWS_PER_W, ROWS_PER_W)],
            sem3,
        )
    for f in range(F_HALF):
        pltpu.make_async_copy(
            x_hbm.at[f, pl.ds(col, ROWS_PER_W)],
            idx_v.at[pl.ds(f * ROWS_PER_W, ROWS_PER_W)],
            sem3,
        ).wait()
    g1 = pltpu.async_copy(
        table_hbm.at[idx_v.at[pl.ds(0, IDX_HALF)]],
        vals_v.at[pl.ds(0, IDX_HALF)],
        sem1,
    )
    for f in range(F_HALF, NUM_FIELDS):
        pltpu.async_copy(
            x_hbm.at[f, pl.ds(col, ROWS_PER_W)],
            idx_v.at[pl.ds(f * ROWS_PER_W, ROWS_PER_W)],
            sem3,
        )
    for f in range(F_HALF, NUM_FIELDS):
        pltpu.make_async_copy(
            x_hbm.at[f, pl.ds(col, ROWS_PER_W)],
            idx_v.at[pl.ds(f * ROWS_PER_W, ROWS_PER_W)],
            sem3,
        ).wait()
    pltpu.sync_copy(bias_hbm, bias_v)
    g2 = pltpu.async_copy(
        table_hbm.at[idx_v.at[pl.ds(IDX_HALF, IDX_PER_W - IDX_HALF)]],
        vals_v.at[pl.ds(IDX_HALF, IDX_PER_W - IDX_HALF)],
        sem2,
    )
    bias_vec = bias_v[...]
    g1.wait()

    def accum1(i, _):
        acc0 = bias_vec
        acc1 = vals_v[pl.ds(i * 16, 16)]
        for f in range(1, F_HALF, 2):
            acc0 = acc0 + vals_v[pl.ds(f * ROWS_PER_W + i * 16, 16)]
        for f in range(2, F_HALF, 2):
            acc1 = acc1 + vals_v[pl.ds(f * ROWS_PER_W + i * 16, 16)]
        out_v[pl.ds(i * 16, 16)] = acc0 + acc1
        return 0

    lax.fori_loop(0, ROWS_PER_W // 16, accum1, 0)
    g2.wait()

    def accum2(i, _):
        acc0 = out_v[pl.ds(i * 16, 16)]
        acc1 = vals_v[pl.ds(F_HALF * ROWS_PER_W + i * 16, 16)]
        for f in range(F_HALF + 1, NUM_FIELDS, 2):
            acc0 = acc0 + vals_v[pl.ds(f * ROWS_PER_W + i * 16, 16)]
        for f in range(F_HALF + 2, NUM_FIELDS, 2):
            acc1 = acc1 + vals_v[pl.ds(f * ROWS_PER_W + i * 16, 16)]
        out_v[pl.ds(i * 16, 16)] = acc0 + acc1
        return 0

    lax.fori_loop(0, ROWS_PER_W // 16, accum2, 0)
    pltpu.sync_copy(out_v, out_hbm.at[pl.ds(wid * ROWS_PER_W, ROWS_PER_W)])


def kernel(x, table, bias):
    table_lin = jnp.concatenate(
        [
            table[:TBL_SPLIT].reshape(-1),
            table[TBL_SPLIT:].reshape(-1),
        ]
    )
    xw = x.T
    out = _emb_sum(xw, table_lin, jnp.broadcast_to(bias, (16,)))
    return out.reshape(BATCH, 1)
